# baseline (speedup 1.0000x reference)
import jax
import jax.numpy as jnp
from jax.experimental import pallas as pl
from jax.experimental.pallas import tpu as pltpu

N_NODES = 10000
D = 128


def _final_body(neigh_ref, w_ref, gamma_ref, beta_ref, out_ref):
    neigh = neigh_ref[...]
    h = jax.lax.dot_general(
        neigh, w_ref[...], (((1,), (0,)), ((), ())),
        precision=jax.lax.Precision.HIGHEST,
        preferred_element_type=jnp.float32)
    mean = jnp.mean(h, axis=0, keepdims=True)
    var = jnp.mean(h * h, axis=0, keepdims=True) - mean * mean
    hn = (h - mean) * jax.lax.rsqrt(var + 1e-5)
    out_ref[...] = jnp.tanh(hn * gamma_ref[...] + beta_ref[...])


def kernel(ent_emb, edge_index, neigh_w, bn_gamma, bn_beta):
    src = edge_index[0]
    dst = edge_index[1]
    norm = jnp.sum(ent_emb[src] * ent_emb[dst], axis=-1)
    seg_max = jax.ops.segment_max(norm, dst, num_segments=N_NODES)
    seg_max = jnp.where(jnp.isfinite(seg_max), seg_max, 0.0)
    e = jnp.exp(norm - seg_max[dst])
    denom = jax.ops.segment_sum(e, dst, num_segments=N_NODES)
    alpha = e / jnp.maximum(denom[dst], 1e-16)
    msg = ent_emb[src] * alpha[:, None]
    neigh = jax.ops.segment_sum(msg, dst, num_segments=N_NODES)

    out = pl.pallas_call(
        _final_body,
        out_shape=jax.ShapeDtypeStruct((N_NODES, D), jnp.float32),
    )(neigh, neigh_w, bn_gamma.reshape(1, D), bn_beta.reshape(1, D))
    return out


# speedup vs baseline: 6.6108x; 6.6108x over previous
import dataclasses
import functools

import jax
import jax.numpy as jnp
from jax import lax
from jax.experimental import pallas as pl
from jax.experimental.pallas import tpu as pltpu
from jax.experimental.pallas import tpu_sc as plsc

N = 10000
NPAD = 10240
D = 128
E = 320000
NC, NS, L = 2, 16, 16
NW = NC * NS
EPW = E // NW
K = 80
NCHUNKS = EPW // K
GRP = K // L
ROWS_PT = NPAD // NS

_mesh = plsc.VectorSubcoreMesh(core_axis_name="c", subcore_axis_name="s")

_sc_params = pltpu.CompilerParams()
if "needs_layout_passes" in pltpu.CompilerParams.__dataclass_fields__:
    _sc_params = dataclasses.replace(_sc_params, needs_layout_passes=False)


def _phase1_body(emb_hbm, sidx_hbm, didx_hbm, norm_hbm, d4_hbm,
                 sidx_v, didx_v, srows, drows, normc, updc, padbuf, zbuf,
                 d4_sp):
    cid = lax.axis_index("c")
    sid = lax.axis_index("s")
    wid = sid * NC + cid

    @pl.loop(0, ROWS_PT, step=L)
    def _(i):
        zbuf[pl.ds(i, L)] = jnp.zeros((L,), jnp.float32)

    pltpu.sync_copy(zbuf, d4_sp.at[pl.ds(sid * ROWS_PT, ROWS_PT)])
    plsc.subcore_barrier()

    base = wid * EPW

    @pl.loop(0, NCHUNKS)
    def _chunk(k):
        off = base + k * K
        pltpu.sync_copy(sidx_hbm.at[pl.ds(off, K)], sidx_v)
        pltpu.sync_copy(didx_hbm.at[pl.ds(off, K)], didx_v)
        pltpu.sync_copy(emb_hbm.at[sidx_v], srows)
        pltpu.sync_copy(emb_hbm.at[didx_v], drows)

        lanes = lax.iota(jnp.int32, L)
        for g in range(GRP):
            for j in range(L):
                e = g * L + j
                acc = srows[e, pl.ds(0, L)] * drows[e, pl.ds(0, L)]
                for ch in range(1, D // L):
                    acc += (srows[e, pl.ds(ch * L, L)]
                            * drows[e, pl.ds(ch * L, L)])
                plsc.store_scatter(padbuf, [lanes + j * (L + 1)], acc)
            norm16 = plsc.load_gather(padbuf, [lanes * (L + 1)])
            for lq in range(1, L):
                norm16 += plsc.load_gather(padbuf, [lanes * (L + 1) + lq])
            normc[pl.ds(g * L, L)] = norm16
            updc[pl.ds(g * L, L)] = jnp.exp(norm16 * 0.25)

        pltpu.sync_copy(normc, norm_hbm.at[pl.ds(off, K)])
        pltpu.sync_copy(updc, d4_sp.at[didx_v], add=True)

    plsc.subcore_barrier()
    sl = pl.ds(sid * ROWS_PT, ROWS_PT)
    pltpu.sync_copy(d4_sp.at[sl], d4_hbm.at[cid, sl])


_phase1 = functools.partial(
    pl.kernel,
    out_type=[jax.ShapeDtypeStruct((E,), jnp.float32),
              jax.ShapeDtypeStruct((NC, NPAD), jnp.float32)],
    mesh=_mesh,
    compiler_params=_sc_params,
    scratch_types=[
        pltpu.VMEM((K,), jnp.int32),
        pltpu.VMEM((K,), jnp.int32),
        pltpu.VMEM((K, D), jnp.float32),
        pltpu.VMEM((K, D), jnp.float32),
        pltpu.VMEM((K,), jnp.float32),
        pltpu.VMEM((K,), jnp.float32),
        pltpu.VMEM((L * (L + 1),), jnp.float32),
        pltpu.VMEM((ROWS_PT,), jnp.float32),
        pltpu.VMEM_SHARED((NPAD,), jnp.float32),
    ],
)(_phase1_body)


def _bcast_lane(v, j):
    idx = jnp.full((L, 1), j, jnp.int32)
    dnums = lax.GatherDimensionNumbers(
        offset_dims=(), collapsed_slice_dims=(0,), start_index_map=(0,))
    return lax.gather(v, idx, dnums, (1,),
                      mode=lax.GatherScatterMode.PROMISE_IN_BOUNDS)


def _phase2_body(emb_hbm, sidx_hbm, didx_hbm, norm_hbm, c_hbm,
                 u_hbm, den_hbm,
                 sidx_v, didx_v, srows, urows, normc, evals, cbuf, zrows,
                 u_sp, den_sp):
    cid = lax.axis_index("c")
    sid = lax.axis_index("s")
    wid = sid * NC + cid

    pltpu.sync_copy(c_hbm, cbuf)

    @pl.loop(0, ROWS_PT, step=L)
    def _(i):
        zrows[pl.ds(i, L)] = jnp.zeros((L,), jnp.float32)

    @pl.loop(0, K, step=1)
    def _(i):
        for ch in range(D // L):
            urows[i, pl.ds(ch * L, L)] = jnp.zeros((L,), jnp.float32)

    rbase = sid * ROWS_PT
    for i in range(ROWS_PT // K):
        pltpu.sync_copy(urows, u_sp.at[pl.ds(rbase + i * K, K)])
    pltpu.sync_copy(zrows, den_sp.at[pl.ds(rbase, ROWS_PT)])
    plsc.subcore_barrier()

    base = wid * EPW

    @pl.loop(0, NCHUNKS)
    def _chunk(k):
        off = base + k * K
        pltpu.sync_copy(sidx_hbm.at[pl.ds(off, K)], sidx_v)
        pltpu.sync_copy(didx_hbm.at[pl.ds(off, K)], didx_v)
        pltpu.sync_copy(norm_hbm.at[pl.ds(off, K)], normc)
        pltpu.sync_copy(emb_hbm.at[sidx_v], srows)

        for g in range(GRP):
            didx16 = didx_v[pl.ds(g * L, L)]
            cvals = plsc.load_gather(cbuf, [didx16])
            e16 = jnp.exp(normc[pl.ds(g * L, L)] - cvals)
            evals[pl.ds(g * L, L)] = e16
            for j in range(L):
                e = g * L + j
                ebc = _bcast_lane(e16, j)
                for ch in range(D // L):
                    urows[e, pl.ds(ch * L, L)] = (
                        srows[e, pl.ds(ch * L, L)] * ebc)

        pltpu.sync_copy(urows, u_sp.at[didx_v], add=True)
        pltpu.sync_copy(evals, den_sp.at[didx_v], add=True)

    plsc.subcore_barrier()
    sl = pl.ds(rbase, ROWS_PT)
    pltpu.sync_copy(u_sp.at[sl], u_hbm.at[cid, sl])
    pltpu.sync_copy(den_sp.at[sl], den_hbm.at[cid, sl])


_phase2 = functools.partial(
    pl.kernel,
    out_type=[jax.ShapeDtypeStruct((NC, NPAD, D), jnp.float32),
              jax.ShapeDtypeStruct((NC, NPAD), jnp.float32)],
    mesh=_mesh,
    compiler_params=_sc_params,
    scratch_types=[
        pltpu.VMEM((K,), jnp.int32),
        pltpu.VMEM((K,), jnp.int32),
        pltpu.VMEM((K, D), jnp.float32),
        pltpu.VMEM((K, D), jnp.float32),
        pltpu.VMEM((K,), jnp.float32),
        pltpu.VMEM((K,), jnp.float32),
        pltpu.VMEM((NPAD,), jnp.float32),
        pltpu.VMEM((ROWS_PT,), jnp.float32),
        pltpu.VMEM_SHARED((NPAD, D), jnp.float32),
        pltpu.VMEM_SHARED((NPAD,), jnp.float32),
    ],
)(_phase2_body)


def _cshift_tc(d4_ref, c_ref):
    s = d4_ref[0] + d4_ref[1]
    c_ref[...] = jnp.where(s > 0.0, 4.0 * jnp.log(jnp.maximum(s, 1e-30)),
                           0.0)


def _final_tc(u_ref, den_ref, w_ref, gamma_ref, beta_ref, out_ref):
    dn = den_ref[0, :N] + den_ref[1, :N]
    un = u_ref[0, :N, :] + u_ref[1, :N, :]
    neigh = un / jnp.maximum(dn, 1e-16)[:, None]
    h = lax.dot_general(neigh, w_ref[...], (((1,), (0,)), ((), ())),
                        precision=lax.Precision.HIGHEST,
                        preferred_element_type=jnp.float32)
    mean = jnp.mean(h, axis=0, keepdims=True)
    var = jnp.mean(h * h, axis=0, keepdims=True) - mean * mean
    hn = (h - mean) * lax.rsqrt(var + 1e-5)
    out_ref[...] = jnp.tanh(hn * gamma_ref[...] + beta_ref[...])


def kernel(ent_emb, edge_index, neigh_w, bn_gamma, bn_beta):
    src = edge_index[0]
    dst = edge_index[1]

    norm, d4 = _phase1(ent_emb, src, dst)

    c = pl.pallas_call(
        _cshift_tc,
        out_shape=jax.ShapeDtypeStruct((NPAD,), jnp.float32),
    )(d4)

    u, den = _phase2(ent_emb, src, dst, norm, c)

    out = pl.pallas_call(
        _final_tc,
        out_shape=jax.ShapeDtypeStruct((N, D), jnp.float32),
    )(u, den, neigh_w, bn_gamma.reshape(1, D), bn_beta.reshape(1, D))
    return out


# speedup vs baseline: 8.8739x; 1.3423x over previous
import dataclasses
import functools

import jax
import jax.numpy as jnp
from jax import lax
from jax.experimental import pallas as pl
from jax.experimental.pallas import tpu as pltpu
from jax.experimental.pallas import tpu_sc as plsc

N = 10000
NPAD = 10240
D = 128
E = 320000
NC, NS, L = 2, 16, 16
NW = NC * NS
EPW = E // NW
K = 80
NCHUNKS = EPW // K
GRP = K // L
ROWS_PT = NPAD // NS

_mesh = plsc.VectorSubcoreMesh(core_axis_name="c", subcore_axis_name="s")

_sc_params = pltpu.CompilerParams()
if "needs_layout_passes" in pltpu.CompilerParams.__dataclass_fields__:
    _sc_params = dataclasses.replace(_sc_params, needs_layout_passes=False)


def _zero16():
    return jnp.zeros((L,), jnp.float32)


def _phase1_body(emb_hbm, sidx_hbm, didx_hbm, norm_hbm, d4_hbm,
                 sidx_a, sidx_b, didx_a, didx_b, dsc_a, dsc_b,
                 srows_a, srows_b, drows_a, drows_b,
                 normc_a, normc_b, updc_a, updc_b, padbuf, zbuf, d4_sp,
                 sem_is0, sem_is1, sem_id0, sem_id1,
                 sem_gs0, sem_gs1, sem_gd0, sem_gd1,
                 sem_on0, sem_on1, sem_oa0, sem_oa1):
    cid = lax.axis_index("c")
    sid = lax.axis_index("s")
    wid = sid * NC + cid

    sidx = (sidx_a, sidx_b)
    didx = (didx_a, didx_b)
    dsc = (dsc_a, dsc_b)
    srows = (srows_a, srows_b)
    drows = (drows_a, drows_b)
    normc = (normc_a, normc_b)
    updc = (updc_a, updc_b)
    sem_is = (sem_is0, sem_is1)
    sem_id = (sem_id0, sem_id1)
    sem_gs = (sem_gs0, sem_gs1)
    sem_gd = (sem_gd0, sem_gd1)
    sem_on = (sem_on0, sem_on1)
    sem_oa = (sem_oa0, sem_oa1)

    @pl.loop(0, ROWS_PT, step=L)
    def _(i):
        zbuf[pl.ds(i, L)] = _zero16()

    pltpu.sync_copy(zbuf, d4_sp.at[pl.ds(sid * ROWS_PT, ROWS_PT)])
    plsc.subcore_barrier()

    base = wid * EPW

    def off_of(ch):
        return jnp.minimum(base + ch * K, E - K)

    def idx_start(p, ch):
        o = off_of(ch)
        pltpu.make_async_copy(
            sidx_hbm.at[pl.ds(o, K)], sidx[p], sem_is[p]).start()
        pltpu.make_async_copy(
            didx_hbm.at[pl.ds(o, K)], didx[p], sem_id[p]).start()

    def idx_wait(p):
        o = off_of(0)
        pltpu.make_async_copy(
            sidx_hbm.at[pl.ds(o, K)], sidx[p], sem_is[p]).wait()
        pltpu.make_async_copy(
            didx_hbm.at[pl.ds(o, K)], didx[p], sem_id[p]).wait()

    def gat_start(p):
        pltpu.make_async_copy(
            emb_hbm.at[sidx[p]], srows[p], sem_gs[p]).start()
        pltpu.make_async_copy(
            emb_hbm.at[didx[p]], drows[p], sem_gd[p]).start()

    def gat_wait(p):
        pltpu.make_async_copy(
            emb_hbm.at[sidx[p]], srows[p], sem_gs[p]).wait()
        pltpu.make_async_copy(
            emb_hbm.at[didx[p]], drows[p], sem_gd[p]).wait()

    def shadow_didx(p):
        for q in range(GRP):
            dsc[p][pl.ds(q * L, L)] = didx[p][pl.ds(q * L, L)]

    def compute(p):
        sr, dr = srows[p], drows[p]
        lanes = lax.iota(jnp.int32, L)
        for g in range(GRP):
            for j in range(L):
                e = g * L + j
                acc = sr[e, pl.ds(0, L)] * dr[e, pl.ds(0, L)]
                for ch in range(1, D // L):
                    acc += sr[e, pl.ds(ch * L, L)] * dr[e, pl.ds(ch * L, L)]
                plsc.store_scatter(padbuf, [lanes + j * (L + 1)], acc)
            norm16 = plsc.load_gather(padbuf, [lanes * (L + 1)])
            for lq in range(1, L):
                norm16 += plsc.load_gather(padbuf, [lanes * (L + 1) + lq])
            normc[p][pl.ds(g * L, L)] = norm16
            updc[p][pl.ds(g * L, L)] = jnp.exp(norm16 * 0.25)

    def outs_start(p, ch):
        o = off_of(ch)
        pltpu.make_async_copy(
            normc[p], norm_hbm.at[pl.ds(o, K)], sem_on[p]).start()
        pltpu.make_async_copy(
            updc[p], d4_sp.at[dsc[p]], sem_oa[p]).start(add=True)

    def outs_wait(p):
        o = off_of(0)
        pltpu.make_async_copy(
            normc[p], norm_hbm.at[pl.ds(o, K)], sem_on[p]).wait()
        pltpu.make_async_copy(
            updc[p], d4_sp.at[dsc[p]], sem_oa[p]).wait()

    idx_start(0, 0)
    idx_start(1, 1)
    idx_wait(0)
    gat_start(0)
    idx_wait(1)
    gat_start(1)

    @pl.loop(0, (NCHUNKS - 1) // 2)
    def _pair(i):
        ch0 = 2 * i
        for p in (0, 1):
            ch = ch0 + p
            gat_wait(p)

            @pl.when(i > 0)
            def _():
                outs_wait(p)

            shadow_didx(p)
            idx_start(p, ch + 2)
            compute(p)
            outs_start(p, ch)
            idx_wait(p)
            gat_start(p)

    gat_wait(0)
    outs_wait(0)
    shadow_didx(0)
    compute(0)
    outs_start(0, NCHUNKS - 1)
    gat_wait(1)
    outs_wait(1)
    outs_wait(0)

    plsc.subcore_barrier()
    sl = pl.ds(sid * ROWS_PT, ROWS_PT)
    pltpu.sync_copy(d4_sp.at[sl], d4_hbm.at[cid, sl])


_phase1 = functools.partial(
    pl.kernel,
    out_type=[jax.ShapeDtypeStruct((E,), jnp.float32),
              jax.ShapeDtypeStruct((NC, NPAD), jnp.float32)],
    mesh=_mesh,
    compiler_params=_sc_params,
    scratch_types=(
        [pltpu.VMEM((K,), jnp.int32)] * 6
        + [pltpu.VMEM((K, D), jnp.float32)] * 4
        + [pltpu.VMEM((K,), jnp.float32)] * 4
        + [pltpu.VMEM((L * (L + 1),), jnp.float32),
           pltpu.VMEM((ROWS_PT,), jnp.float32),
           pltpu.VMEM_SHARED((NPAD,), jnp.float32)]
        + [pltpu.SemaphoreType.DMA] * 12
    ),
)(_phase1_body)


def _bcast_lane(v, j):
    idx = jnp.full((L, 1), j, jnp.int32)
    dnums = lax.GatherDimensionNumbers(
        offset_dims=(), collapsed_slice_dims=(0,), start_index_map=(0,))
    return lax.gather(v, idx, dnums, (1,),
                      mode=lax.GatherScatterMode.PROMISE_IN_BOUNDS)


def _phase2_body(emb_hbm, sidx_hbm, didx_hbm, norm_hbm, c_hbm,
                 u_hbm, den_hbm,
                 sidx_v, didx_v, srows, normc, evals, cbuf, zbuf,
                 u_sp, den_sp):
    cid = lax.axis_index("c")
    sid = lax.axis_index("s")
    wid = sid * NC + cid

    pltpu.sync_copy(c_hbm, cbuf)

    @pl.loop(0, ROWS_PT, step=L)
    def _(i):
        zbuf[pl.ds(i, L)] = _zero16()

    @pl.loop(0, K, step=1)
    def _(i):
        for ch in range(D // L):
            srows[i, pl.ds(ch * L, L)] = _zero16()

    rbase = sid * ROWS_PT
    for i in range(ROWS_PT // K):
        pltpu.sync_copy(srows, u_sp.at[pl.ds(rbase + i * K, K)])
    pltpu.sync_copy(zbuf, den_sp.at[pl.ds(rbase, ROWS_PT)])
    plsc.subcore_barrier()

    base = wid * EPW

    @pl.loop(0, NCHUNKS)
    def _chunk(k):
        off = base + k * K
        pltpu.sync_copy(sidx_hbm.at[pl.ds(off, K)], sidx_v)
        pltpu.sync_copy(didx_hbm.at[pl.ds(off, K)], didx_v)
        pltpu.sync_copy(norm_hbm.at[pl.ds(off, K)], normc)
        pltpu.sync_copy(emb_hbm.at[sidx_v], srows)

        for g in range(GRP):
            didx16 = didx_v[pl.ds(g * L, L)]
            cvals = plsc.load_gather(cbuf, [didx16])
            e16 = jnp.exp(normc[pl.ds(g * L, L)] - cvals)
            evals[pl.ds(g * L, L)] = e16
            for j in range(L):
                e = g * L + j
                ebc = _bcast_lane(e16, j)
                for ch in range(D // L):
                    srows[e, pl.ds(ch * L, L)] = (
                        srows[e, pl.ds(ch * L, L)] * ebc)

        pltpu.sync_copy(srows, u_sp.at[didx_v], add=True)
        pltpu.sync_copy(evals, den_sp.at[didx_v], add=True)

    plsc.subcore_barrier()
    sl = pl.ds(rbase, ROWS_PT)
    pltpu.sync_copy(u_sp.at[sl], u_hbm.at[cid, sl])
    pltpu.sync_copy(den_sp.at[sl], den_hbm.at[cid, sl])


_phase2 = functools.partial(
    pl.kernel,
    out_type=[jax.ShapeDtypeStruct((NC, NPAD, D), jnp.float32),
              jax.ShapeDtypeStruct((NC, NPAD), jnp.float32)],
    mesh=_mesh,
    compiler_params=_sc_params,
    scratch_types=(
        [pltpu.VMEM((K,), jnp.int32)] * 2
        + [pltpu.VMEM((K, D), jnp.float32)]
        + [pltpu.VMEM((K,), jnp.float32)] * 2
        + [pltpu.VMEM((NPAD,), jnp.float32),
           pltpu.VMEM((ROWS_PT,), jnp.float32),
           pltpu.VMEM_SHARED((NPAD, D), jnp.float32),
           pltpu.VMEM_SHARED((NPAD,), jnp.float32)]
    ),
)(_phase2_body)


def _cshift_tc(d4_ref, c_ref):
    s = d4_ref[0] + d4_ref[1]
    c_ref[...] = jnp.where(s > 0.0, 4.0 * jnp.log(jnp.maximum(s, 1e-30)),
                           0.0)


def _final_tc(u_ref, den_ref, w_ref, gamma_ref, beta_ref, out_ref):
    dn = den_ref[0, :N] + den_ref[1, :N]
    un = u_ref[0, :N, :] + u_ref[1, :N, :]
    neigh = un / jnp.maximum(dn, 1e-16)[:, None]
    h = lax.dot_general(neigh, w_ref[...], (((1,), (0,)), ((), ())),
                        precision=lax.Precision.HIGHEST,
                        preferred_element_type=jnp.float32)
    mean = jnp.mean(h, axis=0, keepdims=True)
    var = jnp.mean(h * h, axis=0, keepdims=True) - mean * mean
    hn = (h - mean) * lax.rsqrt(var + 1e-5)
    out_ref[...] = jnp.tanh(hn * gamma_ref[...] + beta_ref[...])


def kernel(ent_emb, edge_index, neigh_w, bn_gamma, bn_beta):
    src = edge_index[0]
    dst = edge_index[1]

    norm, d4 = _phase1(ent_emb, src, dst)

    c = pl.pallas_call(
        _cshift_tc,
        out_shape=jax.ShapeDtypeStruct((NPAD,), jnp.float32),
    )(d4)

    u, den = _phase2(ent_emb, src, dst, norm, c)

    out = pl.pallas_call(
        _final_tc,
        out_shape=jax.ShapeDtypeStruct((N, D), jnp.float32),
    )(u, den, neigh_w, bn_gamma.reshape(1, D), bn_beta.reshape(1, D))
    return out


# speedup vs baseline: 13.1535x; 1.4823x over previous
import dataclasses
import functools

import jax
import jax.numpy as jnp
from jax import lax
from jax.experimental import pallas as pl
from jax.experimental.pallas import tpu as pltpu
from jax.experimental.pallas import tpu_sc as plsc

N = 10000
NPAD = 10240
D = 128
E = 320000
NC, NS, L = 2, 16, 16
NW = NC * NS
EPW = E // NW
K = 80
NCHUNKS = EPW // K
GRP = K // L
ROWS_PT = NPAD // NS

_mesh = plsc.VectorSubcoreMesh(core_axis_name="c", subcore_axis_name="s")

_sc_params = pltpu.CompilerParams()
if "needs_layout_passes" in pltpu.CompilerParams.__dataclass_fields__:
    _sc_params = dataclasses.replace(_sc_params, needs_layout_passes=False)


def _zero16():
    return jnp.zeros((L,), jnp.float32)


def _phase1_body(emb_hbm, sidx_hbm, didx_hbm, norm_hbm, d4_hbm,
                 sidx_a, sidx_b, didx_a, didx_b, dsc_a, dsc_b,
                 srows_a, srows_b, drows_a, drows_b,
                 normc_a, normc_b, updc_a, updc_b, padbuf, zbuf, d4_sp,
                 sem_is0, sem_is1, sem_id0, sem_id1,
                 sem_gs0, sem_gs1, sem_gd0, sem_gd1,
                 sem_on0, sem_on1, sem_oa0, sem_oa1):
    cid = lax.axis_index("c")
    sid = lax.axis_index("s")
    wid = sid * NC + cid

    sidx = (sidx_a, sidx_b)
    didx = (didx_a, didx_b)
    dsc = (dsc_a, dsc_b)
    srows = (srows_a, srows_b)
    drows = (drows_a, drows_b)
    normc = (normc_a, normc_b)
    updc = (updc_a, updc_b)
    sem_is = (sem_is0, sem_is1)
    sem_id = (sem_id0, sem_id1)
    sem_gs = (sem_gs0, sem_gs1)
    sem_gd = (sem_gd0, sem_gd1)
    sem_on = (sem_on0, sem_on1)
    sem_oa = (sem_oa0, sem_oa1)

    @pl.loop(0, ROWS_PT, step=L)
    def _(i):
        zbuf[pl.ds(i, L)] = _zero16()

    pltpu.sync_copy(zbuf, d4_sp.at[pl.ds(sid * ROWS_PT, ROWS_PT)])
    plsc.subcore_barrier()

    base = wid * EPW

    def off_of(ch):
        return jnp.minimum(base + ch * K, E - K)

    def idx_start(p, ch):
        o = off_of(ch)
        pltpu.make_async_copy(
            sidx_hbm.at[pl.ds(o, K)], sidx[p], sem_is[p]).start()
        pltpu.make_async_copy(
            didx_hbm.at[pl.ds(o, K)], didx[p], sem_id[p]).start()

    def idx_wait(p):
        o = off_of(0)
        pltpu.make_async_copy(
            sidx_hbm.at[pl.ds(o, K)], sidx[p], sem_is[p]).wait()
        pltpu.make_async_copy(
            didx_hbm.at[pl.ds(o, K)], didx[p], sem_id[p]).wait()

    def gat_start(p):
        pltpu.make_async_copy(
            emb_hbm.at[sidx[p]], srows[p], sem_gs[p]).start()
        pltpu.make_async_copy(
            emb_hbm.at[didx[p]], drows[p], sem_gd[p]).start()

    def gat_wait(p):
        pltpu.make_async_copy(
            emb_hbm.at[sidx[p]], srows[p], sem_gs[p]).wait()
        pltpu.make_async_copy(
            emb_hbm.at[didx[p]], drows[p], sem_gd[p]).wait()

    def shadow_didx(p):
        for q in range(GRP):
            dsc[p][pl.ds(q * L, L)] = didx[p][pl.ds(q * L, L)]

    def compute(p):
        sr, dr = srows[p], drows[p]
        lanes = lax.iota(jnp.int32, L)
        for g in range(GRP):
            for j in range(L):
                e = g * L + j
                acc = sr[e, pl.ds(0, L)] * dr[e, pl.ds(0, L)]
                for ch in range(1, D // L):
                    acc += sr[e, pl.ds(ch * L, L)] * dr[e, pl.ds(ch * L, L)]
                plsc.store_scatter(padbuf, [lanes + j * (L + 1)], acc)
            norm16 = plsc.load_gather(padbuf, [lanes * (L + 1)])
            for lq in range(1, L):
                norm16 += plsc.load_gather(padbuf, [lanes * (L + 1) + lq])
            normc[p][pl.ds(g * L, L)] = norm16
            updc[p][pl.ds(g * L, L)] = jnp.exp(norm16 * 0.25)

    def outs_start(p, ch):
        o = off_of(ch)
        pltpu.make_async_copy(
            normc[p], norm_hbm.at[pl.ds(o, K)], sem_on[p]).start()
        pltpu.make_async_copy(
            updc[p], d4_sp.at[dsc[p]], sem_oa[p]).start(add=True)

    def outs_wait(p):
        o = off_of(0)
        pltpu.make_async_copy(
            normc[p], norm_hbm.at[pl.ds(o, K)], sem_on[p]).wait()
        pltpu.make_async_copy(
            updc[p], d4_sp.at[dsc[p]], sem_oa[p]).wait()

    idx_start(0, 0)
    idx_start(1, 1)
    idx_wait(0)
    gat_start(0)
    idx_wait(1)
    gat_start(1)

    @pl.loop(0, (NCHUNKS - 1) // 2)
    def _pair(i):
        ch0 = 2 * i
        for p in (0, 1):
            ch = ch0 + p
            gat_wait(p)

            @pl.when(i > 0)
            def _():
                outs_wait(p)

            shadow_didx(p)
            idx_start(p, ch + 2)
            compute(p)
            outs_start(p, ch)
            idx_wait(p)
            gat_start(p)

    gat_wait(0)
    outs_wait(0)
    shadow_didx(0)
    compute(0)
    outs_start(0, NCHUNKS - 1)
    gat_wait(1)
    outs_wait(1)
    outs_wait(0)

    plsc.subcore_barrier()
    sl = pl.ds(sid * ROWS_PT, ROWS_PT)
    pltpu.sync_copy(d4_sp.at[sl], d4_hbm.at[cid, sl])


_phase1 = functools.partial(
    pl.kernel,
    out_type=[jax.ShapeDtypeStruct((E,), jnp.float32),
              jax.ShapeDtypeStruct((NC, NPAD), jnp.float32)],
    mesh=_mesh,
    compiler_params=_sc_params,
    scratch_types=(
        [pltpu.VMEM((K,), jnp.int32)] * 6
        + [pltpu.VMEM((K, D), jnp.float32)] * 4
        + [pltpu.VMEM((K,), jnp.float32)] * 4
        + [pltpu.VMEM((L * (L + 1),), jnp.float32),
           pltpu.VMEM((ROWS_PT,), jnp.float32),
           pltpu.VMEM_SHARED((NPAD,), jnp.float32)]
        + [pltpu.SemaphoreType.DMA] * 12
    ),
)(_phase1_body)


def _bcast_lane(v, j):
    idx = jnp.full((L, 1), j, jnp.int32)
    dnums = lax.GatherDimensionNumbers(
        offset_dims=(), collapsed_slice_dims=(0,), start_index_map=(0,))
    return lax.gather(v, idx, dnums, (1,),
                      mode=lax.GatherScatterMode.PROMISE_IN_BOUNDS)


def _phase2_body(emb_hbm, sidx_hbm, didx_hbm, norm_hbm, c_hbm,
                 u_hbm, den_hbm,
                 sidx_a, sidx_b, didx_a, didx_b, dsc_a, dsc_b,
                 srows_a, srows_b, urows_a, urows_b,
                 normc_a, normc_b, evals_a, evals_b, cvals_a, cvals_b, zbuf,
                 u_sp, den_sp,
                 sem_is0, sem_is1, sem_id0, sem_id1, sem_in0, sem_in1,
                 sem_gs0, sem_gs1, sem_gc0, sem_gc1,
                 sem_ou0, sem_ou1, sem_od0, sem_od1):
    cid = lax.axis_index("c")
    sid = lax.axis_index("s")
    wid = sid * NC + cid

    sidx = (sidx_a, sidx_b)
    didx = (didx_a, didx_b)
    dsc = (dsc_a, dsc_b)
    srows = (srows_a, srows_b)
    urows = (urows_a, urows_b)
    normc = (normc_a, normc_b)
    evals = (evals_a, evals_b)
    cvals = (cvals_a, cvals_b)
    sem_is = (sem_is0, sem_is1)
    sem_id = (sem_id0, sem_id1)
    sem_in = (sem_in0, sem_in1)
    sem_gs = (sem_gs0, sem_gs1)
    sem_gc = (sem_gc0, sem_gc1)
    sem_ou = (sem_ou0, sem_ou1)
    sem_od = (sem_od0, sem_od1)

    rbase = sid * ROWS_PT

    @pl.loop(0, ROWS_PT, step=L)
    def _(i):
        zbuf[pl.ds(i, L)] = _zero16()

    @pl.loop(0, K, step=1)
    def _(i):
        for ch in range(D // L):
            urows_a[i, pl.ds(ch * L, L)] = _zero16()

    for i in range(ROWS_PT // K):
        pltpu.sync_copy(urows_a, u_sp.at[pl.ds(rbase + i * K, K)])
    pltpu.sync_copy(zbuf, den_sp.at[pl.ds(rbase, ROWS_PT)])
    plsc.subcore_barrier()

    base = wid * EPW

    def off_of(ch):
        return jnp.minimum(base + ch * K, E - K)

    def idx_start(p, ch):
        o = off_of(ch)
        pltpu.make_async_copy(
            sidx_hbm.at[pl.ds(o, K)], sidx[p], sem_is[p]).start()
        pltpu.make_async_copy(
            didx_hbm.at[pl.ds(o, K)], didx[p], sem_id[p]).start()

    def idx_wait(p):
        o = off_of(0)
        pltpu.make_async_copy(
            sidx_hbm.at[pl.ds(o, K)], sidx[p], sem_is[p]).wait()
        pltpu.make_async_copy(
            didx_hbm.at[pl.ds(o, K)], didx[p], sem_id[p]).wait()

    def norm_start(p, ch):
        o = off_of(ch)
        pltpu.make_async_copy(
            norm_hbm.at[pl.ds(o, K)], normc[p], sem_in[p]).start()

    def norm_wait(p):
        o = off_of(0)
        pltpu.make_async_copy(
            norm_hbm.at[pl.ds(o, K)], normc[p], sem_in[p]).wait()

    def gat_start(p):
        pltpu.make_async_copy(
            emb_hbm.at[sidx[p]], srows[p], sem_gs[p]).start()
        pltpu.make_async_copy(
            c_hbm.at[didx[p]], cvals[p], sem_gc[p]).start()

    def gat_wait(p):
        pltpu.make_async_copy(
            emb_hbm.at[sidx[p]], srows[p], sem_gs[p]).wait()
        pltpu.make_async_copy(
            c_hbm.at[didx[p]], cvals[p], sem_gc[p]).wait()

    def shadow_didx(p):
        for q in range(GRP):
            dsc[p][pl.ds(q * L, L)] = didx[p][pl.ds(q * L, L)]

    def compute(p):
        sr, ur = srows[p], urows[p]
        for g in range(GRP):
            e16 = jnp.exp(normc[p][pl.ds(g * L, L)]
                          - cvals[p][pl.ds(g * L, L)])
            evals[p][pl.ds(g * L, L)] = e16
            for j in range(L):
                e = g * L + j
                ebc = _bcast_lane(e16, j)
                for ch in range(D // L):
                    ur[e, pl.ds(ch * L, L)] = sr[e, pl.ds(ch * L, L)] * ebc

    def outs_start(p):
        pltpu.make_async_copy(
            urows[p], u_sp.at[dsc[p]], sem_ou[p]).start(add=True)
        pltpu.make_async_copy(
            evals[p], den_sp.at[dsc[p]], sem_od[p]).start(add=True)

    def outs_wait(p):
        pltpu.make_async_copy(
            urows[p], u_sp.at[dsc[p]], sem_ou[p]).wait()
        pltpu.make_async_copy(
            evals[p], den_sp.at[dsc[p]], sem_od[p]).wait()

    idx_start(0, 0)
    idx_start(1, 1)
    norm_start(0, 0)
    norm_start(1, 1)
    idx_wait(0)
    gat_start(0)
    idx_wait(1)
    gat_start(1)

    @pl.loop(0, (NCHUNKS - 1) // 2)
    def _pair(i):
        for p in (0, 1):
            ch = 2 * i + p
            gat_wait(p)
            norm_wait(p)

            @pl.when(i > 0)
            def _():
                outs_wait(p)

            shadow_didx(p)
            idx_start(p, ch + 2)
            compute(p)
            outs_start(p)
            norm_start(p, ch + 2)
            idx_wait(p)
            gat_start(p)

    gat_wait(0)
    norm_wait(0)
    outs_wait(0)
    shadow_didx(0)
    compute(0)
    outs_start(0)
    gat_wait(1)
    norm_wait(1)
    outs_wait(1)
    outs_wait(0)

    plsc.subcore_barrier()
    sl = pl.ds(rbase, ROWS_PT)
    pltpu.sync_copy(u_sp.at[sl], u_hbm.at[cid, sl])
    pltpu.sync_copy(den_sp.at[sl], den_hbm.at[cid, sl])


_phase2 = functools.partial(
    pl.kernel,
    out_type=[jax.ShapeDtypeStruct((NC, NPAD, D), jnp.float32),
              jax.ShapeDtypeStruct((NC, NPAD), jnp.float32)],
    mesh=_mesh,
    compiler_params=_sc_params,
    scratch_types=(
        [pltpu.VMEM((K,), jnp.int32)] * 6
        + [pltpu.VMEM((K, D), jnp.float32)] * 4
        + [pltpu.VMEM((K,), jnp.float32)] * 6
        + [pltpu.VMEM((ROWS_PT,), jnp.float32),
           pltpu.VMEM_SHARED((NPAD, D), jnp.float32),
           pltpu.VMEM_SHARED((NPAD,), jnp.float32)]
        + [pltpu.SemaphoreType.DMA] * 14
    ),
)(_phase2_body)


def _cshift_tc(d4_ref, c_ref):
    s = d4_ref[0] + d4_ref[1]
    c_ref[...] = jnp.where(s > 0.0, 4.0 * jnp.log(jnp.maximum(s, 1e-30)),
                           0.0)


def _final_tc(u_ref, den_ref, w_ref, gamma_ref, beta_ref, out_ref):
    dn = den_ref[0, :N] + den_ref[1, :N]
    un = u_ref[0, :N, :] + u_ref[1, :N, :]
    neigh = un / jnp.maximum(dn, 1e-16)[:, None]
    h = lax.dot_general(neigh, w_ref[...], (((1,), (0,)), ((), ())),
                        precision=lax.Precision.HIGHEST,
                        preferred_element_type=jnp.float32)
    mean = jnp.mean(h, axis=0, keepdims=True)
    var = jnp.mean(h * h, axis=0, keepdims=True) - mean * mean
    hn = (h - mean) * lax.rsqrt(var + 1e-5)
    out_ref[...] = jnp.tanh(hn * gamma_ref[...] + beta_ref[...])


def kernel(ent_emb, edge_index, neigh_w, bn_gamma, bn_beta):
    src = edge_index[0]
    dst = edge_index[1]

    norm, d4 = _phase1(ent_emb, src, dst)

    c = pl.pallas_call(
        _cshift_tc,
        out_shape=jax.ShapeDtypeStruct((NPAD,), jnp.float32),
    )(d4)

    u, den = _phase2(ent_emb, src, dst, norm, c)

    out = pl.pallas_call(
        _final_tc,
        out_shape=jax.ShapeDtypeStruct((N, D), jnp.float32),
    )(u, den, neigh_w, bn_gamma.reshape(1, D), bn_beta.reshape(1, D))
    return out
